# baseline (speedup 1.0000x reference)
import functools

import jax
import jax.numpy as jnp
from jax import lax
from jax.experimental import pallas as pl
from jax.experimental.pallas import tpu as pltpu
from jax.experimental.pallas import tpu_sc as plsc

VOCAB = 10000
D = 128
T = 10
B = 16384

NC = 2
NS = 16
NW = NC * NS
BPW = B // NW
CHUNK = 128
NCHUNK = BPW // CHUNK
L = 16

_SLAB = 624
_LAST = VOCAB - (NS - 1) * _SLAB


def _body(idx_hbm, table_hbm, out_hbm, table_sp, idxt_v, acc_v,
          wsem0, wsem1, asem0, asem1, osem0, osem1):
    sid = lax.axis_index("s")
    wid = sid * NC + lax.axis_index("c")
    base = wid * BPW
    wsems = [wsem0, wsem1]
    asems = [asem0, asem1]
    osems = [osem0, osem1]

    pltpu.sync_copy(idx_hbm.at[:, pl.ds(base, BPW)], idxt_v)

    def fire_write(c, src):
        buf = c % 2
        co = c * CHUNK
        pltpu.async_copy(
            src.at[idxt_v.at[0, pl.ds(co, CHUNK)]], acc_v.at[buf], wsems[buf]
        )

    fire_write(0, table_hbm)
    fire_write(1, table_hbm)

    slab = jnp.where(sid == NS - 1, _LAST, _SLAB)
    pltpu.sync_copy(
        table_hbm.at[pl.ds(sid * _SLAB, slab), :],
        table_sp.at[pl.ds(sid * _SLAB, slab), :],
    )
    plsc.subcore_barrier()

    def fire_adds(c):
        buf = c % 2
        co = c * CHUNK
        pltpu.make_async_copy(
            table_hbm.at[pl.ds(0, CHUNK), :], acc_v.at[buf], wsems[buf]
        ).wait()

        @pl.loop(1, T)
        def _(t):
            pltpu.async_copy(
                table_sp.at[idxt_v.at[t, pl.ds(co, CHUNK)]],
                acc_v.at[buf],
                asems[buf],
                add=True,
            )

    scale = jnp.float32(1.0 / T)

    def drain_ship(c):
        buf = c % 2
        co = c * CHUNK

        for _ in range(1, T):
            pltpu.make_async_copy(
                table_hbm.at[pl.ds(0, CHUNK), :], acc_v.at[buf], asems[buf]
            ).wait()

        @pl.loop(0, CHUNK, unroll=4)
        def _(i):
            for d in range(D // L):
                acc_v[buf, i, pl.ds(d * L, L)] = acc_v[buf, i, pl.ds(d * L, L)] * scale

        pltpu.async_copy(
            acc_v.at[buf], out_hbm.at[pl.ds(base + co, CHUNK), :], osems[buf]
        )

    fire_adds(0)
    fire_adds(1)
    for c in range(NCHUNK):
        drain_ship(c)
        if c + 2 < NCHUNK:
            buf = c % 2
            pltpu.make_async_copy(
                acc_v.at[buf], out_hbm.at[pl.ds(base, CHUNK), :], osems[buf]
            ).wait()
            fire_write(c + 2, table_sp)
            fire_adds(c + 2)
    for buf in range(2):
        pltpu.make_async_copy(
            acc_v.at[buf], out_hbm.at[pl.ds(base, CHUNK), :], osems[buf]
        ).wait()


@jax.jit
def _pooled_lookup(idx_t, table):
    mesh = plsc.VectorSubcoreMesh(core_axis_name="c", subcore_axis_name="s")
    return pl.kernel(
        _body,
        out_type=jax.ShapeDtypeStruct((B, D), jnp.float32),
        mesh=mesh,
        scratch_types=[
            pltpu.VMEM_SHARED((VOCAB, D), jnp.float32),
            pltpu.VMEM((T, BPW), jnp.int32),
            pltpu.VMEM((2, CHUNK, D), jnp.float32),
            pltpu.SemaphoreType.DMA,
            pltpu.SemaphoreType.DMA,
            pltpu.SemaphoreType.DMA,
            pltpu.SemaphoreType.DMA,
            pltpu.SemaphoreType.DMA,
            pltpu.SemaphoreType.DMA,
        ],
    )(idx_t, table)


def kernel(text, embedding):
    idx_t = text.astype(jnp.int32).T
    return _pooled_lookup(idx_t, embedding)


# speedup vs baseline: 1.0079x; 1.0079x over previous
import functools

import jax
import jax.numpy as jnp
from jax import lax
from jax.experimental import pallas as pl
from jax.experimental.pallas import tpu as pltpu
from jax.experimental.pallas import tpu_sc as plsc

VOCAB = 10000
D = 128
T = 10
B = 16384

NC = 2
NS = 16
NW = NC * NS
BPW = B // NW
CHUNK = 128
NCHUNK = BPW // CHUNK
L = 16

_SLAB = 624
_LAST = VOCAB - (NS - 1) * _SLAB


def _body(idx_hbm, table_hbm, out_hbm, table_sp, idxt_v, acc_v,
          wsem0, wsem1, asem0, asem1, osem0, osem1):
    sid = lax.axis_index("s")
    wid = sid * NC + lax.axis_index("c")
    base = wid * BPW
    wsems = [wsem0, wsem1]
    asems = [asem0, asem1]
    osems = [osem0, osem1]

    pltpu.sync_copy(idx_hbm.at[:, pl.ds(base, BPW)], idxt_v)

    def fire_write(c, src):
        buf = c % 2
        co = c * CHUNK
        pltpu.async_copy(
            src.at[idxt_v.at[0, pl.ds(co, CHUNK)]], acc_v.at[buf], wsems[buf]
        )

    fire_write(0, table_hbm)
    fire_write(1, table_hbm)

    slab = jnp.where(sid == NS - 1, _LAST, _SLAB)
    pltpu.sync_copy(
        table_hbm.at[pl.ds(sid * _SLAB, slab), :],
        table_sp.at[pl.ds(sid * _SLAB, slab), :],
    )
    plsc.subcore_barrier()

    def fire_adds(c):
        buf = c % 2
        co = c * CHUNK
        pltpu.make_async_copy(
            table_hbm.at[pl.ds(0, CHUNK), :], acc_v.at[buf], wsems[buf]
        ).wait()

        @pl.loop(1, T)
        def _(t):
            pltpu.async_copy(
                table_sp.at[idxt_v.at[t, pl.ds(co, CHUNK)]],
                acc_v.at[buf],
                asems[buf],
                add=True,
            )

    scale = jnp.float32(1.0 / T)

    def drain_ship(c):
        buf = c % 2
        co = c * CHUNK

        @pl.loop(1, T)
        def _(t):
            pltpu.make_async_copy(
                table_hbm.at[pl.ds(0, CHUNK), :], acc_v.at[buf], asems[buf]
            ).wait()

        @pl.loop(0, CHUNK)
        def _(i):
            for d in range(D // L):
                acc_v[buf, i, pl.ds(d * L, L)] = acc_v[buf, i, pl.ds(d * L, L)] * scale

        pltpu.async_copy(
            acc_v.at[buf], out_hbm.at[pl.ds(base + co, CHUNK), :], osems[buf]
        )

    fire_adds(0)
    fire_adds(1)
    for c in range(NCHUNK):
        drain_ship(c)
        if c + 2 < NCHUNK:
            buf = c % 2
            pltpu.make_async_copy(
                acc_v.at[buf], out_hbm.at[pl.ds(base, CHUNK), :], osems[buf]
            ).wait()
            fire_write(c + 2, table_sp)
            fire_adds(c + 2)
    for buf in range(2):
        pltpu.make_async_copy(
            acc_v.at[buf], out_hbm.at[pl.ds(base, CHUNK), :], osems[buf]
        ).wait()


@jax.jit
def _pooled_lookup(idx_t, table):
    mesh = plsc.VectorSubcoreMesh(core_axis_name="c", subcore_axis_name="s")
    return pl.kernel(
        _body,
        out_type=jax.ShapeDtypeStruct((B, D), jnp.float32),
        mesh=mesh,
        scratch_types=[
            pltpu.VMEM_SHARED((VOCAB, D), jnp.float32),
            pltpu.VMEM((T, BPW), jnp.int32),
            pltpu.VMEM((2, CHUNK, D), jnp.float32),
            pltpu.SemaphoreType.DMA,
            pltpu.SemaphoreType.DMA,
            pltpu.SemaphoreType.DMA,
            pltpu.SemaphoreType.DMA,
            pltpu.SemaphoreType.DMA,
            pltpu.SemaphoreType.DMA,
        ],
    )(idx_t, table)


def kernel(text, embedding):
    idx_t = text.astype(jnp.int32).T
    return _pooled_lookup(idx_t, embedding)
